# baseline (speedup 1.0000x reference)
import functools

import jax
import jax.numpy as jnp
from jax import lax
from jax.experimental import pallas as pl
from jax.experimental.pallas import tpu as pltpu
from jax.experimental.pallas import tpu_sc as plsc

CB = 128
D = 32
BLOCK = 2048
SEQ = 8

NC = 2
NS = 16
NW = NC * NS
REP = 4
ROW = CB * D


def _vq_body(lat_ref, cb_ref, q_ref, p_ref):
    x = lat_ref[...]
    cb = cb_ref[...]
    s = jnp.sum(x * x, axis=1, keepdims=True)
    n = jnp.sum(cb * cb, axis=1)[None, :]
    mm = jax.lax.dot_general(x, cb, (((1,), (1,)), ((), ())),
                             preferred_element_type=jnp.float32)
    d = (s + n) - 2.0 * mm
    dmin = jnp.min(d, axis=1, keepdims=True)
    lane = jax.lax.broadcasted_iota(jnp.int32, (BLOCK, CB), 1)
    idx = jnp.min(jnp.where(d == dmin, lane, CB), axis=1, keepdims=True)
    oh = (lane == idx).astype(jnp.float32)
    q = jax.lax.dot_general(oh, cb, (((1,), (0,)), ((), ())),
                            preferred_element_type=jnp.float32)
    q_ref[...] = q
    p_ref[...] = q


def _cbs_sc_body(cb_hbm, out_hbm, buf, sem):
    wid = lax.axis_index("s") * NC + lax.axis_index("c")
    b_per_w = out_hbm.shape[0] // NW
    for j in range(REP):
        pltpu.sync_copy(cb_hbm, buf.at[j])
    base = wid * b_per_w
    n_bursts = b_per_w // REP
    for i0 in range(0, n_bursts, 8):
        handles = [
            pltpu.async_copy(
                buf, out_hbm.at[pl.ds(base + (i0 + i) * REP, REP)], sem)
            for i in range(min(8, n_bursts - i0))
        ]
        for h in handles:
            h.wait()


def _make_cbs(B):
    return functools.partial(
        pl.kernel,
        out_type=jax.ShapeDtypeStruct((B, CB, D), jnp.float32),
        mesh=plsc.VectorSubcoreMesh(core_axis_name="c", subcore_axis_name="s"),
        scratch_types=[
            pltpu.VMEM((REP, CB, D), jnp.float32),
            pltpu.SemaphoreType.DMA,
        ],
    )(_cbs_sc_body)


def kernel(latent, codebook):
    B = latent.shape[0]
    rows = B * SEQ
    lat2 = latent.reshape(rows, D)
    grid = rows // BLOCK
    q, p = pl.pallas_call(
        _vq_body,
        grid=(grid,),
        in_specs=[
            pl.BlockSpec((BLOCK, D), lambda i: (i, 0)),
            pl.BlockSpec((CB, D), lambda i: (0, 0)),
        ],
        out_specs=[
            pl.BlockSpec((BLOCK, D), lambda i: (i, 0)),
            pl.BlockSpec((BLOCK, D), lambda i: (i, 0)),
        ],
        out_shape=[
            jax.ShapeDtypeStruct((rows, D), jnp.float32),
            jax.ShapeDtypeStruct((rows, D), jnp.float32),
        ],
    )(lat2, codebook)
    cbs = _make_cbs(B)(codebook)
    shape = latent.shape
    return (p.reshape(shape), q.reshape(shape), cbs)


# speedup vs baseline: 4.0534x; 4.0534x over previous
import jax
import jax.numpy as jnp
from jax.experimental import pallas as pl

CB = 128
D = 32
BLOCK = 4096
SEQ = 8


def _vq_body(lat_ref, cb_ref, lane_ref, q_ref, p_ref):
    x = lat_ref[...]
    cb = cb_ref[...]
    s = jnp.sum(x * x, axis=1, keepdims=True)
    n = jnp.sum(cb * cb, axis=1)[None, :]
    mm = jax.lax.dot_general(x, cb, (((1,), (1,)), ((), ())),
                             preferred_element_type=jnp.float32)
    d = (s + n) - 2.0 * mm
    dmin = jnp.min(d, axis=1, keepdims=True)
    lane = lane_ref[...]
    idx = jnp.min(jnp.where(d == dmin, lane, float(CB)), axis=1, keepdims=True)
    oh = (lane == idx).astype(jnp.float32)
    q = jax.lax.dot_general(oh, cb, (((1,), (0,)), ((), ())),
                            preferred_element_type=jnp.float32)
    q_ref[...] = q
    p_ref[...] = q


def kernel(latent, codebook):
    B = latent.shape[0]
    rows = B * SEQ
    lat2 = latent.reshape(rows, D)
    grid = rows // BLOCK
    q, p = pl.pallas_call(
        _vq_body,
        grid=(grid,),
        in_specs=[
            pl.BlockSpec((BLOCK, D), lambda i: (i, 0)),
            pl.BlockSpec((CB, D), lambda i: (0, 0)),
            pl.BlockSpec((1, CB), lambda i: (0, 0)),
        ],
        out_specs=[
            pl.BlockSpec((BLOCK, D), lambda i: (i, 0)),
            pl.BlockSpec((BLOCK, D), lambda i: (i, 0)),
        ],
        out_shape=[
            jax.ShapeDtypeStruct((rows, D), jnp.float32),
            jax.ShapeDtypeStruct((rows, D), jnp.float32),
        ],
    )(lat2, codebook, jnp.arange(CB, dtype=jnp.float32).reshape(1, CB))
    cbs = jnp.broadcast_to(codebook[None], (B, CB, D))
    shape = latent.shape
    return (p.reshape(shape), q.reshape(shape), cbs)


# speedup vs baseline: 4.2288x; 1.0433x over previous
import jax
import jax.numpy as jnp
from jax.experimental import pallas as pl

CB = 128
D = 32
BLOCK = 8192
SEQ = 8


def _vq_body(lat_ref, cb_ref, lane_ref, q_ref, p_ref):
    x = lat_ref[...]
    cb = cb_ref[...]
    s = jnp.sum(x * x, axis=1, keepdims=True)
    n = jnp.sum(cb * cb, axis=1)[None, :]
    mm = jax.lax.dot_general(x, cb, (((1,), (1,)), ((), ())),
                             preferred_element_type=jnp.float32)
    d = (s + n) - 2.0 * mm
    dmin = jnp.min(d, axis=1, keepdims=True)
    lane = lane_ref[...]
    idx = jnp.min(jnp.where(d == dmin, lane, float(CB)), axis=1, keepdims=True)
    oh = (lane == idx).astype(jnp.float32)
    q = jax.lax.dot_general(oh, cb, (((1,), (0,)), ((), ())),
                            preferred_element_type=jnp.float32)
    q_ref[...] = q
    p_ref[...] = q


def kernel(latent, codebook):
    B = latent.shape[0]
    rows = B * SEQ
    lat2 = latent.reshape(rows, D)
    grid = rows // BLOCK
    q, p = pl.pallas_call(
        _vq_body,
        grid=(grid,),
        in_specs=[
            pl.BlockSpec((BLOCK, D), lambda i: (i, 0)),
            pl.BlockSpec((CB, D), lambda i: (0, 0)),
            pl.BlockSpec((1, CB), lambda i: (0, 0)),
        ],
        out_specs=[
            pl.BlockSpec((BLOCK, D), lambda i: (i, 0)),
            pl.BlockSpec((BLOCK, D), lambda i: (i, 0)),
        ],
        out_shape=[
            jax.ShapeDtypeStruct((rows, D), jnp.float32),
            jax.ShapeDtypeStruct((rows, D), jnp.float32),
        ],
    )(lat2, codebook, jnp.arange(CB, dtype=jnp.float32).reshape(1, CB))
    cbs = jnp.broadcast_to(codebook[None], (B, CB, D))
    shape = latent.shape
    return (p.reshape(shape), q.reshape(shape), cbs)
